# baseline (speedup 1.0000x reference)
import functools

import jax
import jax.numpy as jnp
from jax import lax
from jax.experimental import pallas as pl
from jax.experimental.pallas import tpu as pltpu
from jax.experimental.pallas import tpu_sc as plsc

_B, _S, _H = 256, 512, 768
_EPS = 1e-12
_L = 16
_NH = _H // _L
_C = 32


def _sc_embed_ln(ids, tts, word, pos2, tdiff, gamma, beta):
    info = plsc.get_sparse_core_info()
    nw = info.num_cores * info.num_subcores
    tok = ids.shape[0]
    rows_per_w = _B // nw
    npc = _S // _C

    mesh = plsc.VectorSubcoreMesh(core_axis_name="c", subcore_axis_name="s")

    @functools.partial(
        pl.kernel,
        mesh=mesh,
        out_type=jax.ShapeDtypeStruct((tok, _H), jnp.float32),
        compiler_params=pltpu.CompilerParams(needs_layout_passes=False),
        scratch_types=[
            pltpu.VMEM((_C,), jnp.int32),
            pltpu.VMEM((_C,), jnp.int32),
            pltpu.VMEM((_C, _H), jnp.float32),
            pltpu.VMEM((_C, _H), jnp.float32),
            pltpu.VMEM((_H,), jnp.float32),
            pltpu.VMEM((_H,), jnp.float32),
            pltpu.VMEM((_H,), jnp.float32),
            pltpu.SemaphoreType.DMA,
        ],
    )
    def k(ids_h, tts_h, word_h, pos_h, diff_h, gamma_h, beta_h, out_h,
          idx_v, tt_v, rows_v, pos_v, diff_v, gamma_v, beta_v, sem):
        wid = lax.axis_index("s") * info.num_cores + lax.axis_index("c")
        pltpu.sync_copy(diff_h, diff_v)
        pltpu.sync_copy(gamma_h, gamma_v)
        pltpu.sync_copy(beta_h, beta_v)

        def pc_body(pc, _):
            pltpu.sync_copy(pos_h.at[pl.ds(pc * _C, _C)], pos_v)

            def b_body(b, _):
                base = (wid * rows_per_w + b) * _S + pc * _C
                pltpu.sync_copy(ids_h.at[pl.ds(base, _C)], idx_v)
                pltpu.sync_copy(tts_h.at[pl.ds(base, _C)], tt_v)
                pltpu.async_copy(word_h.at[idx_v], rows_v, sem).wait()

                def tok_body(i, _):
                    ivec = jnp.full((_L,), i, jnp.int32)
                    tf = plsc.load_gather(tt_v, [ivec]).astype(jnp.float32)
                    acc_s = jnp.zeros((_L,), jnp.float32)
                    acc_q = jnp.zeros((_L,), jnp.float32)
                    for j in range(_NH):
                        sl = pl.ds(j * _L, _L)
                        v = rows_v[i, sl] + pos_v[i, sl] + tf * diff_v[sl]
                        rows_v[i, sl] = v
                        acc_s = acc_s + v
                        acc_q = acc_q + v * v
                    mean = jnp.sum(acc_s) * (1.0 / _H)
                    var = jnp.sum(acc_q) * (1.0 / _H) - mean * mean
                    x = jnp.full((_L,), var + _EPS, jnp.float32)
                    xi = lax.bitcast_convert_type(x, jnp.int32)
                    yi = jnp.int32(0x5F3759DF) - lax.shift_right_logical(xi, 1)
                    y = lax.bitcast_convert_type(yi, jnp.float32)
                    for _n in range(3):
                        y = y * (1.5 - 0.5 * x * y * y)
                    mv = jnp.full((_L,), mean, jnp.float32)
                    for j in range(_NH):
                        sl = pl.ds(j * _L, _L)
                        v = rows_v[i, sl]
                        rows_v[i, sl] = (v - mv) * y * gamma_v[sl] + beta_v[sl]
                    return None

                lax.fori_loop(0, _C, tok_body, None)
                pltpu.sync_copy(rows_v, out_h.at[pl.ds(base, _C)])
                return None

            lax.fori_loop(0, rows_per_w, b_body, None)
            return None

        lax.fori_loop(0, npc, pc_body, None)

    return k(ids, tts, word, pos2, tdiff, gamma, beta)


def kernel(input_ids, token_type_ids, word_emb, pos_emb, type_emb, gamma, beta):
    ids = input_ids.reshape(-1).astype(jnp.int32)
    tts = token_type_ids.reshape(-1).astype(jnp.int32)
    pos2 = pos_emb + type_emb[0]
    tdiff = type_emb[1] - type_emb[0]
    out = _sc_embed_ln(ids, tts, word_emb, pos2, tdiff, gamma, beta)
    return out.reshape(_B, _S, _H)
